# baseline (speedup 1.0000x reference)
import functools

import jax
import jax.numpy as jnp
from jax import lax
from jax.experimental import pallas as pl
from jax.experimental.pallas import tpu as pltpu
from jax.experimental.pallas import tpu_sc as plsc

NUM_TASKS = 100000
PROMPT_DIM = 128
BATCH = 16384

_NC = 2
_NS = 16
_NW = _NC * _NS
_CHUNK = 128
_B_PER_W = BATCH // _NW
_CH_PER_W = _B_PER_W // _CHUNK


def _gather_body(idx_hbm, table_hbm, out_hbm, idx_v, rows_v, gsem, wsem):
    wid = lax.axis_index("s") * _NC + lax.axis_index("c")
    row0 = wid * _CH_PER_W
    pltpu.sync_copy(idx_hbm.at[pl.ds(row0, _CH_PER_W)], idx_v)
    gathers = [None] * _CH_PER_W
    writes = []
    for j in range(2):
        gathers[j] = pltpu.async_copy(
            table_hbm.at[idx_v.at[j]], rows_v.at[j], gsem.at[j])
    for j in range(_CH_PER_W):
        gathers[j].wait()
        writes.append(pltpu.async_copy(rows_v.at[j], out_hbm.at[row0 + j], wsem))
        if j + 2 < _CH_PER_W:
            gathers[j + 2] = pltpu.async_copy(
                table_hbm.at[idx_v.at[j + 2]], rows_v.at[j + 2], gsem.at[j + 2])
    for c in writes:
        c.wait()


_sc_gather = pl.kernel(
    _gather_body,
    out_type=jax.ShapeDtypeStruct((BATCH // _CHUNK, _CHUNK, PROMPT_DIM),
                                  jnp.float32),
    mesh=plsc.VectorSubcoreMesh(core_axis_name="c", subcore_axis_name="s"),
    scratch_types=[
        pltpu.VMEM((_CH_PER_W, _CHUNK), jnp.int32),
        pltpu.VMEM((_CH_PER_W, _CHUNK, PROMPT_DIM), jnp.float32),
        pltpu.SemaphoreType.DMA((_CH_PER_W,)),
        pltpu.SemaphoreType.DMA,
    ],
)


@jax.jit
def kernel(task_id, table):
    idx = task_id.astype(jnp.int32).reshape(BATCH // _CHUNK, _CHUNK)
    out = _sc_gather(idx, table)
    return out.reshape(BATCH, PROMPT_DIM)


# speedup vs baseline: 1.0340x; 1.0340x over previous
import functools

import jax
import jax.numpy as jnp
from jax import lax
from jax.experimental import pallas as pl
from jax.experimental.pallas import tpu as pltpu
from jax.experimental.pallas import tpu_sc as plsc

NUM_TASKS = 100000
PROMPT_DIM = 128
BATCH = 16384

_NC = 2
_NS = 16
_NW = _NC * _NS
_CHUNK = 128
_B_PER_W = BATCH // _NW
_CH_PER_W = _B_PER_W // _CHUNK


def _gather_body(idx_hbm, table_hbm, out_hbm, idx_v, rows_v, gsem, wsem):
    wid = lax.axis_index("s") * _NC + lax.axis_index("c")
    row0 = wid * _CH_PER_W
    pltpu.sync_copy(idx_hbm.at[pl.ds(row0, _CH_PER_W)], idx_v)
    half = _CH_PER_W // 2
    gathers = [
        pltpu.async_copy(table_hbm.at[idx_v.at[j]], rows_v.at[j], gsem.at[j])
        for j in range(_CH_PER_W)
    ]
    for j in range(half):
        gathers[j].wait()
    w0 = pltpu.async_copy(
        rows_v.at[pl.ds(0, half)], out_hbm.at[pl.ds(row0, half)], wsem)
    for j in range(half, _CH_PER_W):
        gathers[j].wait()
    w1 = pltpu.async_copy(
        rows_v.at[pl.ds(half, half)], out_hbm.at[pl.ds(row0 + half, half)], wsem)
    w0.wait()
    w1.wait()


_sc_gather = pl.kernel(
    _gather_body,
    out_type=jax.ShapeDtypeStruct((BATCH // _CHUNK, _CHUNK, PROMPT_DIM),
                                  jnp.float32),
    mesh=plsc.VectorSubcoreMesh(core_axis_name="c", subcore_axis_name="s"),
    scratch_types=[
        pltpu.VMEM((_CH_PER_W, _CHUNK), jnp.int32),
        pltpu.VMEM((_CH_PER_W, _CHUNK, PROMPT_DIM), jnp.float32),
        pltpu.SemaphoreType.DMA((_CH_PER_W,)),
        pltpu.SemaphoreType.DMA,
    ],
)


@jax.jit
def kernel(task_id, table):
    idx = task_id.astype(jnp.int32).reshape(BATCH // _CHUNK, _CHUNK)
    out = _sc_gather(idx, table)
    return out.reshape(BATCH, PROMPT_DIM)


# speedup vs baseline: 1.0432x; 1.0088x over previous
import jax
import jax.numpy as jnp
from jax import lax
from jax.experimental import pallas as pl
from jax.experimental.pallas import tpu as pltpu
from jax.experimental.pallas import tpu_sc as plsc

NUM_TASKS = 100000
PROMPT_DIM = 128
BATCH = 16384

_NC = 2
_NS = 16
_NW = _NC * _NS
_CHUNK = 128
_B_PER_W = BATCH // _NW
_CH_PER_W = _B_PER_W // _CHUNK


def _gather_body(idx_hbm, table_hbm, out_hbm, idx_v, rows_v, sem):
    wid = lax.axis_index("s") * _NC + lax.axis_index("c")
    row0 = wid * _CH_PER_W
    pltpu.sync_copy(idx_hbm.at[pl.ds(row0, _CH_PER_W)], idx_v)
    gathers = [
        pltpu.async_copy(table_hbm.at[idx_v.at[j]], rows_v.at[j], sem)
        for j in range(_CH_PER_W)
    ]
    for g in gathers:
        g.wait()
    pltpu.sync_copy(rows_v, out_hbm.at[pl.ds(row0, _CH_PER_W)])


_sc_gather = pl.kernel(
    _gather_body,
    out_type=jax.ShapeDtypeStruct((BATCH // _CHUNK, _CHUNK, PROMPT_DIM),
                                  jnp.float32),
    mesh=plsc.VectorSubcoreMesh(core_axis_name="c", subcore_axis_name="s"),
    scratch_types=[
        pltpu.VMEM((_CH_PER_W, _CHUNK), jnp.int32),
        pltpu.VMEM((_CH_PER_W, _CHUNK, PROMPT_DIM), jnp.float32),
        pltpu.SemaphoreType.DMA,
    ],
)


@jax.jit
def kernel(task_id, table):
    idx = task_id.astype(jnp.int32).reshape(BATCH // _CHUNK, _CHUNK)
    out = _sc_gather(idx, table)
    return out.reshape(BATCH, PROMPT_DIM)
